# baseline (speedup 1.0000x reference)
import functools

import jax
import jax.numpy as jnp
from jax import lax
from jax.experimental import pallas as pl
from jax.experimental.pallas import tpu as pltpu
from jax.experimental.pallas import tpu_sc as plsc

N = 10000
K = 32
D = 128

M_SC = 4800
N_TC = N - M_SC
SC_ROW0 = N_TC * K

NW = 32
NPW = M_SC // NW
C = 15
CHUNKS = NPW // C
RC = C * K

BLOCK = 400


def _sc_sums(nx1d):
    mesh = plsc.VectorSubcoreMesh(core_axis_name="c", subcore_axis_name="s")

    @functools.partial(
        pl.kernel,
        out_type=jax.ShapeDtypeStruct((M_SC * D,), jnp.float32),
        mesh=mesh,
        scratch_types=[
            pltpu.VMEM((RC * D,), jnp.float32),
            pltpu.VMEM((C * D,), jnp.float32),
        ],
    )
    def sc_kernel(nx_ref, out_ref, buf, obuf):
        wid = lax.axis_index("s") * 2 + lax.axis_index("c")

        def chunk(g, carry):
            node0 = wid * NPW + g * C
            pltpu.sync_copy(
                nx_ref.at[pl.ds((SC_ROW0 + node0 * K) * D, RC * D)], buf
            )
            for c in range(C):
                def kb(k, accs):
                    return tuple(
                        accs[d] + buf[pl.ds((c * K + k) * D + d * 16, 16)]
                        for d in range(8)
                    )
                accs = lax.fori_loop(
                    0, K, kb,
                    tuple(jnp.zeros((16,), jnp.float32) for _ in range(8)),
                )
                for d in range(8):
                    obuf[pl.ds(c * D + d * 16, 16)] = accs[d]
            pltpu.sync_copy(obuf, out_ref.at[pl.ds(node0 * D, C * D)])
            return carry

        lax.fori_loop(0, CHUNKS, chunk, 0)

    return sc_kernel(nx1d)


def _tc_fused_body(x_ref, n_ref, wl_ref, wr_ref, b_ref, o_ref):
    mean = jnp.mean(n_ref[...].reshape(BLOCK, K, D), axis=1)
    acc = jnp.dot(x_ref[...], wl_ref[...], preferred_element_type=jnp.float32)
    acc = acc + jnp.dot(mean, wr_ref[...], preferred_element_type=jnp.float32)
    o_ref[...] = acc + b_ref[...]


def _tc_combine_body(x_ref, s_ref, wl_ref, wr_ref, b_ref, o_ref):
    acc = jnp.dot(x_ref[...], wl_ref[...], preferred_element_type=jnp.float32)
    acc = acc + jnp.dot(s_ref[...], wr_ref[...], preferred_element_type=jnp.float32)
    o_ref[...] = acc + b_ref[...]


def kernel(x, neigh_x, W_l, b_l, W_r, b_r):
    wl_t = W_l.T
    wr_t = W_r.T
    bias = (b_l + b_r).reshape(1, D)
    nx2d = neigh_x.reshape(N * K, D)

    sums = _sc_sums(neigh_x.reshape(-1)).reshape(M_SC, D)

    out_tc = pl.pallas_call(
        _tc_fused_body,
        grid=(N_TC // BLOCK,),
        in_specs=[
            pl.BlockSpec((BLOCK, D), lambda i: (i, 0)),
            pl.BlockSpec((BLOCK * K, D), lambda i: (i, 0)),
            pl.BlockSpec((D, D), lambda i: (0, 0)),
            pl.BlockSpec((D, D), lambda i: (0, 0)),
            pl.BlockSpec((1, D), lambda i: (0, 0)),
        ],
        out_specs=pl.BlockSpec((BLOCK, D), lambda i: (i, 0)),
        out_shape=jax.ShapeDtypeStruct((N_TC, D), jnp.float32),
    )(x, nx2d, wl_t, wr_t, bias)

    off = N_TC // BLOCK
    out_sc = pl.pallas_call(
        _tc_combine_body,
        grid=(M_SC // BLOCK,),
        in_specs=[
            pl.BlockSpec((BLOCK, D), lambda i: (i + off, 0)),
            pl.BlockSpec((BLOCK, D), lambda i: (i, 0)),
            pl.BlockSpec((D, D), lambda i: (0, 0)),
            pl.BlockSpec((D, D), lambda i: (0, 0)),
            pl.BlockSpec((1, D), lambda i: (0, 0)),
        ],
        out_specs=pl.BlockSpec((BLOCK, D), lambda i: (i, 0)),
        out_shape=jax.ShapeDtypeStruct((M_SC, D), jnp.float32),
    )(x, sums, wl_t, wr_t * (1.0 / K), bias)

    return jnp.concatenate([out_tc, out_sc], axis=0)


# speedup vs baseline: 1.0781x; 1.0781x over previous
import functools

import jax
import jax.numpy as jnp
from jax import lax
from jax.experimental import pallas as pl
from jax.experimental.pallas import tpu as pltpu
from jax.experimental.pallas import tpu_sc as plsc

N = 10000
K = 32
D = 128

M_SC = 4800
N_TC = N - M_SC
SC_ROW0 = N_TC * K

NW = 32
NPW = M_SC // NW
C = 15
CHUNKS = NPW // C
RC = C * K

BLOCK = 400


def _sc_sums(nx1d):
    mesh = plsc.VectorSubcoreMesh(core_axis_name="c", subcore_axis_name="s")

    @functools.partial(
        pl.kernel,
        out_type=jax.ShapeDtypeStruct((M_SC * D,), jnp.float32),
        mesh=mesh,
        scratch_types=[
            pltpu.VMEM((RC * D,), jnp.float32),
            pltpu.VMEM((RC * D,), jnp.float32),
            pltpu.VMEM((C * D,), jnp.float32),
            pltpu.SemaphoreType.DMA,
            pltpu.SemaphoreType.DMA,
        ],
    )
    def sc_kernel(nx_ref, out_ref, buf_a, buf_b, obuf, sem_a, sem_b):
        wid = lax.axis_index("s") * 2 + lax.axis_index("c")
        base = wid * NPW

        def src(g):
            return nx_ref.at[pl.ds((SC_ROW0 + (base + g * C) * K) * D, RC * D)]

        def reduce_chunk(g, buf):
            def node(c, carry):
                accs = [buf[pl.ds(c * K * D + d * 16, 16)] for d in range(8)]
                for k in range(1, K):
                    for d in range(8):
                        accs[d] = accs[d] + buf[
                            pl.ds(c * K * D + k * D + d * 16, 16)
                        ]
                for d in range(8):
                    obuf[pl.ds(c * D + d * 16, 16)] = accs[d]
                return carry

            lax.fori_loop(0, C, node, 0)
            pltpu.sync_copy(obuf, out_ref.at[pl.ds((base + g * C) * D, C * D)])

        pltpu.async_copy(src(0), buf_a, sem_a)

        def pair(p, carry):
            g0 = 2 * p
            pltpu.make_async_copy(src(0), buf_a, sem_a).wait()

            @pl.when(g0 + 1 < CHUNKS)
            def _():
                pltpu.async_copy(src(g0 + 1), buf_b, sem_b)

            reduce_chunk(g0, buf_a)

            @pl.when(g0 + 1 < CHUNKS)
            def _():
                pltpu.make_async_copy(src(0), buf_b, sem_b).wait()

                @pl.when(g0 + 2 < CHUNKS)
                def _():
                    pltpu.async_copy(src(g0 + 2), buf_a, sem_a)

                reduce_chunk(g0 + 1, buf_b)

            return carry

        lax.fori_loop(0, (CHUNKS + 1) // 2, pair, 0)

    return sc_kernel(nx1d)


def _tc_fused_body(x_ref, n_ref, wl_ref, wr_ref, b_ref, o_ref):
    mean = jnp.mean(n_ref[...].reshape(BLOCK, K, D), axis=1)
    acc = jnp.dot(x_ref[...], wl_ref[...], preferred_element_type=jnp.float32)
    acc = acc + jnp.dot(mean, wr_ref[...], preferred_element_type=jnp.float32)
    o_ref[...] = acc + b_ref[...]


def _tc_combine_body(x_ref, s_ref, wl_ref, wr_ref, b_ref, o_ref):
    acc = jnp.dot(x_ref[...], wl_ref[...], preferred_element_type=jnp.float32)
    acc = acc + jnp.dot(s_ref[...], wr_ref[...], preferred_element_type=jnp.float32)
    o_ref[...] = acc + b_ref[...]


def kernel(x, neigh_x, W_l, b_l, W_r, b_r):
    wl_t = W_l.T
    wr_t = W_r.T
    bias = (b_l + b_r).reshape(1, D)
    nx2d = neigh_x.reshape(N * K, D)

    sums = _sc_sums(neigh_x.reshape(-1)).reshape(M_SC, D)

    out_tc = pl.pallas_call(
        _tc_fused_body,
        grid=(N_TC // BLOCK,),
        in_specs=[
            pl.BlockSpec((BLOCK, D), lambda i: (i, 0)),
            pl.BlockSpec((BLOCK * K, D), lambda i: (i, 0)),
            pl.BlockSpec((D, D), lambda i: (0, 0)),
            pl.BlockSpec((D, D), lambda i: (0, 0)),
            pl.BlockSpec((1, D), lambda i: (0, 0)),
        ],
        out_specs=pl.BlockSpec((BLOCK, D), lambda i: (i, 0)),
        out_shape=jax.ShapeDtypeStruct((N_TC, D), jnp.float32),
    )(x, nx2d, wl_t, wr_t, bias)

    off = N_TC // BLOCK
    out_sc = pl.pallas_call(
        _tc_combine_body,
        grid=(M_SC // BLOCK,),
        in_specs=[
            pl.BlockSpec((BLOCK, D), lambda i: (i + off, 0)),
            pl.BlockSpec((BLOCK, D), lambda i: (i, 0)),
            pl.BlockSpec((D, D), lambda i: (0, 0)),
            pl.BlockSpec((D, D), lambda i: (0, 0)),
            pl.BlockSpec((1, D), lambda i: (0, 0)),
        ],
        out_specs=pl.BlockSpec((BLOCK, D), lambda i: (i, 0)),
        out_shape=jax.ShapeDtypeStruct((M_SC, D), jnp.float32),
    )(x, sums, wl_t, wr_t * (1.0 / K), bias)

    return jnp.concatenate([out_tc, out_sc], axis=0)


# speedup vs baseline: 1.1214x; 1.0401x over previous
import functools

import jax
import jax.numpy as jnp
from jax import lax
from jax.experimental import pallas as pl
from jax.experimental.pallas import tpu as pltpu
from jax.experimental.pallas import tpu_sc as plsc

N = 10000
K = 32
D = 128

M_SC = 4800
N_TC = N - M_SC
SC_ROW0 = N_TC * K

NW = 32
NPW = M_SC // NW
C = 10
CHUNKS = NPW // C
RC = C * K

BLOCK = 400


def _sc_sums(nx1d):
    mesh = plsc.VectorSubcoreMesh(core_axis_name="c", subcore_axis_name="s")

    @functools.partial(
        pl.kernel,
        out_type=jax.ShapeDtypeStruct((M_SC * D,), jnp.float32),
        mesh=mesh,
        scratch_types=[
            pltpu.VMEM((RC * D,), jnp.float32),
            pltpu.VMEM((RC * D,), jnp.float32),
            pltpu.VMEM((NPW * D,), jnp.float32),
            pltpu.SemaphoreType.DMA,
            pltpu.SemaphoreType.DMA,
        ],
    )
    def sc_kernel(nx_ref, out_ref, buf_a, buf_b, obuf, sem_a, sem_b):
        wid = lax.axis_index("s") * 2 + lax.axis_index("c")
        base = wid * NPW

        def src(g):
            return nx_ref.at[pl.ds((SC_ROW0 + (base + g * C) * K) * D, RC * D)]

        def reduce_chunk(g, buf):
            def node(c, carry):
                accs = [buf[pl.ds(c * K * D + d * 16, 16)] for d in range(8)]
                for k in range(1, K):
                    for d in range(8):
                        accs[d] = accs[d] + buf[
                            pl.ds(c * K * D + k * D + d * 16, 16)
                        ]
                for d in range(8):
                    obuf[pl.ds((g * C + c) * D + d * 16, 16)] = accs[d]
                return carry

            lax.fori_loop(0, C, node, 0)

        pltpu.async_copy(src(0), buf_a, sem_a)

        def pair(p, carry):
            g0 = 2 * p
            pltpu.make_async_copy(src(0), buf_a, sem_a).wait()

            @pl.when(g0 + 1 < CHUNKS)
            def _():
                pltpu.async_copy(src(g0 + 1), buf_b, sem_b)

            reduce_chunk(g0, buf_a)

            @pl.when(g0 + 1 < CHUNKS)
            def _():
                pltpu.make_async_copy(src(0), buf_b, sem_b).wait()

                @pl.when(g0 + 2 < CHUNKS)
                def _():
                    pltpu.async_copy(src(g0 + 2), buf_a, sem_a)

                reduce_chunk(g0 + 1, buf_b)

            return carry

        lax.fori_loop(0, (CHUNKS + 1) // 2, pair, 0)
        pltpu.sync_copy(obuf, out_ref.at[pl.ds(base * D, NPW * D)])

    return sc_kernel(nx1d)


def _tc_fused_body(x_ref, n_ref, wl_ref, wr_ref, b_ref, o_ref):
    mean = jnp.mean(n_ref[...].reshape(BLOCK, K, D), axis=1)
    acc = jnp.dot(x_ref[...], wl_ref[...], preferred_element_type=jnp.float32)
    acc = acc + jnp.dot(mean, wr_ref[...], preferred_element_type=jnp.float32)
    o_ref[...] = acc + b_ref[...]


def _tc_combine_body(x_ref, s_ref, wl_ref, wr_ref, b_ref, o_ref):
    acc = jnp.dot(x_ref[...], wl_ref[...], preferred_element_type=jnp.float32)
    acc = acc + jnp.dot(s_ref[...], wr_ref[...], preferred_element_type=jnp.float32)
    o_ref[...] = acc + b_ref[...]


def kernel(x, neigh_x, W_l, b_l, W_r, b_r):
    wl_t = W_l.T
    wr_t = W_r.T
    bias = (b_l + b_r).reshape(1, D)
    nx2d = neigh_x.reshape(N * K, D)

    sums = _sc_sums(neigh_x.reshape(-1)).reshape(M_SC, D)

    out_tc = pl.pallas_call(
        _tc_fused_body,
        grid=(N_TC // BLOCK,),
        in_specs=[
            pl.BlockSpec((BLOCK, D), lambda i: (i, 0)),
            pl.BlockSpec((BLOCK * K, D), lambda i: (i, 0)),
            pl.BlockSpec((D, D), lambda i: (0, 0)),
            pl.BlockSpec((D, D), lambda i: (0, 0)),
            pl.BlockSpec((1, D), lambda i: (0, 0)),
        ],
        out_specs=pl.BlockSpec((BLOCK, D), lambda i: (i, 0)),
        out_shape=jax.ShapeDtypeStruct((N_TC, D), jnp.float32),
    )(x, nx2d, wl_t, wr_t, bias)

    off = N_TC // BLOCK
    out_sc = pl.pallas_call(
        _tc_combine_body,
        grid=(M_SC // BLOCK,),
        in_specs=[
            pl.BlockSpec((BLOCK, D), lambda i: (i + off, 0)),
            pl.BlockSpec((BLOCK, D), lambda i: (i, 0)),
            pl.BlockSpec((D, D), lambda i: (0, 0)),
            pl.BlockSpec((D, D), lambda i: (0, 0)),
            pl.BlockSpec((1, D), lambda i: (0, 0)),
        ],
        out_specs=pl.BlockSpec((BLOCK, D), lambda i: (i, 0)),
        out_shape=jax.ShapeDtypeStruct((M_SC, D), jnp.float32),
    )(x, sums, wl_t, wr_t * (1.0 / K), bias)

    return jnp.concatenate([out_tc, out_sc], axis=0)


# speedup vs baseline: 1.7355x; 1.5476x over previous
import jax
import jax.numpy as jnp
from jax.experimental import pallas as pl

N = 10000
K = 32
D = 128
BLOCK = 400


def _body(x_ref, n_ref, wl_ref, wr_ref, b_ref, o_ref):
    mean = jnp.mean(n_ref[...], axis=1)
    acc = jnp.dot(x_ref[...], wl_ref[...], preferred_element_type=jnp.float32)
    acc = acc + jnp.dot(mean, wr_ref[...], preferred_element_type=jnp.float32)
    o_ref[...] = acc + b_ref[...]


def kernel(x, neigh_x, W_l, b_l, W_r, b_r):
    wl_t = W_l.T
    wr_t = W_r.T
    bias = (b_l + b_r).reshape(1, D)
    return pl.pallas_call(
        _body,
        grid=(N // BLOCK,),
        in_specs=[
            pl.BlockSpec((BLOCK, D), lambda i: (i, 0)),
            pl.BlockSpec((BLOCK, K, D), lambda i: (i, 0, 0)),
            pl.BlockSpec((D, D), lambda i: (0, 0)),
            pl.BlockSpec((D, D), lambda i: (0, 0)),
            pl.BlockSpec((1, D), lambda i: (0, 0)),
        ],
        out_specs=pl.BlockSpec((BLOCK, D), lambda i: (i, 0)),
        out_shape=jax.ShapeDtypeStruct((N, D), jnp.float32),
    )(x, neigh_x, wl_t, wr_t, bias)


# speedup vs baseline: 1.7388x; 1.0019x over previous
import jax
import jax.numpy as jnp
from jax.experimental import pallas as pl

N = 10000
K = 32
D = 128
BLOCK = 400


def _body(x_ref, n_ref, wl_ref, wr_ref, b_ref, o_ref):
    mean = jnp.mean(n_ref[...].reshape(BLOCK, K, D), axis=1)
    acc = jnp.dot(x_ref[...], wl_ref[...], preferred_element_type=jnp.float32)
    acc = acc + jnp.dot(mean, wr_ref[...], preferred_element_type=jnp.float32)
    o_ref[...] = acc + b_ref[...]


def kernel(x, neigh_x, W_l, b_l, W_r, b_r):
    wl_t = W_l.T
    wr_t = W_r.T
    bias = (b_l + b_r).reshape(1, D)
    return pl.pallas_call(
        _body,
        grid=(N // BLOCK,),
        in_specs=[
            pl.BlockSpec((BLOCK, D), lambda i: (i, 0)),
            pl.BlockSpec((BLOCK * K, D), lambda i: (i, 0)),
            pl.BlockSpec((D, D), lambda i: (0, 0)),
            pl.BlockSpec((D, D), lambda i: (0, 0)),
            pl.BlockSpec((1, D), lambda i: (0, 0)),
        ],
        out_specs=pl.BlockSpec((BLOCK, D), lambda i: (i, 0)),
        out_shape=jax.ShapeDtypeStruct((N, D), jnp.float32),
    )(x, neigh_x.reshape(N * K, D), wl_t, wr_t, bias)
